# baseline (speedup 1.0000x reference)
import functools

import jax
import jax.numpy as jnp
from jax import lax
from jax.experimental import pallas as pl
from jax.experimental.pallas import tpu as pltpu
from jax.experimental.pallas import tpu_sc as plsc

_B = 16384
_K = 1024
_D = 256
_R = 2048
_COMMITMENT_COST = 0.25

_NW = 32
_RPW = _B // _NW
_CH = 32
_NCHUNK = _RPW // _CH
_NSC = _RPW // 128


def _vq_tc_kernel(x_ref, sx_ref, se_ref, w_ref,
                  idx_ref, q_ref, cnt_ref, ds_ref):
    x = x_ref[...]
    w = w_ref[...]
    m = jax.lax.dot_general(x, w, (((1,), (1,)), ((), ())),
                            preferred_element_type=jnp.float32)
    d = (sx_ref[...] + se_ref[...]) - 2.0 * m
    dmin = jnp.min(d, axis=1, keepdims=True)
    cols = jax.lax.broadcasted_iota(jnp.int32, (d.shape[0], d.shape[1]), 1)
    idx = jnp.min(jnp.where(d == dmin, cols, _K), axis=1, keepdims=True)
    idx_ref[...] = idx
    onehot = (cols == idx).astype(jnp.float32)
    q = jax.lax.dot_general(onehot, w, (((1,), (0,)), ((), ())),
                            preferred_element_type=jnp.float32)
    q_ref[...] = x + (q - x)
    ones = jnp.ones((1, x.shape[0]), jnp.float32)
    cnt = jax.lax.dot_general(ones, onehot, (((1,), (0,)), ((), ())),
                              preferred_element_type=jnp.float32)
    cnt_ref[...] = cnt.reshape(1, 1, _K)
    ds_ref[...] = jnp.sum(dmin).reshape(1, 1, 1)


def _sc_enc_body(idx_hbm, enc_hbm,
                 idx_v, zbuf, ones_v, f0, f1, f2, f3,
                 semz, sems):
    wid = lax.axis_index("s") * 2 + lax.axis_index("c")
    base = wid * _RPW
    pltpu.sync_copy(idx_hbm.at[pl.ds(base, _RPW)], idx_v)

    def _zero(t, carry):
        zbuf[pl.ds(t * 16, 16)] = jnp.zeros((16,), jnp.float32)
        return carry

    lax.fori_loop(0, _CH * _K // 16, _zero, 0)
    for t in range(128 // 16):
        ones_v[pl.ds(t * 16, 16)] = jnp.ones((16,), jnp.float32)

    lanes = lax.iota(jnp.int32, 16)
    fbufs = (f0, f1, f2, f3)
    for g in range(_RPW // 16):
        colv = idx_v[pl.ds(g * 16, 16)]
        rows = (base + g * 16) + lanes
        fbufs[g // 8][pl.ds((g % 8) * 16, 16)] = rows * _K + colv

    zcopies = []
    for c in range(_NCHUNK):
        zcopies.append(pltpu.async_copy(
            zbuf, enc_hbm.at[pl.ds((base + c * _CH) * _K, _CH * _K)], semz))
    for zc in zcopies:
        zc.wait()
    scopies = []
    for j in range(_NSC):
        scopies.append(pltpu.async_copy(ones_v, enc_hbm.at[fbufs[j]], sems))
    for sc in scopies:
        sc.wait()


def kernel(inputs, classes, embeddings_weight):
    del classes
    input_shape = inputs.shape
    x = inputs.reshape(_B, _D)
    sx = jnp.sum(x ** 2, axis=1, keepdims=True)
    se = jnp.sum(embeddings_weight ** 2, axis=1)[None, :]
    grid = _B // _R
    idx, q, cnt, ds = pl.pallas_call(
        _vq_tc_kernel,
        grid=(grid,),
        in_specs=[
            pl.BlockSpec((_R, _D), lambda i: (i, 0)),
            pl.BlockSpec((_R, 1), lambda i: (i, 0)),
            pl.BlockSpec((1, _K), lambda i: (0, 0)),
            pl.BlockSpec((_K, _D), lambda i: (0, 0)),
        ],
        out_specs=[
            pl.BlockSpec((_R, 1), lambda i: (i, 0)),
            pl.BlockSpec((_R, _D), lambda i: (i, 0)),
            pl.BlockSpec((1, 1, _K), lambda i: (i, 0, 0)),
            pl.BlockSpec((1, 1, 1), lambda i: (i, 0, 0)),
        ],
        out_shape=[
            jax.ShapeDtypeStruct((_B, 1), jnp.int32),
            jax.ShapeDtypeStruct((_B, _D), jnp.float32),
            jax.ShapeDtypeStruct((grid, 1, _K), jnp.float32),
            jax.ShapeDtypeStruct((grid, 1, 1), jnp.float32),
        ],
        compiler_params=pltpu.CompilerParams(
            dimension_semantics=("parallel",)),
    )(x, sx, se, embeddings_weight)

    sc_enc = functools.partial(
        pl.kernel,
        mesh=plsc.VectorSubcoreMesh(core_axis_name="c", subcore_axis_name="s"),
        out_type=jax.ShapeDtypeStruct((_B * _K,), jnp.float32),
        scratch_types=[
            pltpu.VMEM((_RPW,), jnp.int32),
            pltpu.VMEM((_CH * _K,), jnp.float32),
            pltpu.VMEM((128,), jnp.float32),
            pltpu.VMEM((128,), jnp.int32),
            pltpu.VMEM((128,), jnp.int32),
            pltpu.VMEM((128,), jnp.int32),
            pltpu.VMEM((128,), jnp.int32),
            pltpu.SemaphoreType.DMA,
            pltpu.SemaphoreType.DMA,
        ],
    )(_sc_enc_body)
    enc_flat = sc_enc(idx.reshape(_B))

    loss = (1.0 + _COMMITMENT_COST) * jnp.sum(ds) / (_B * _D)
    p = jnp.sum(cnt.reshape(grid, _K), axis=0) / _B
    perp = jnp.exp(-jnp.sum(p * jnp.log(p + 1e-10)))
    return (loss, q.reshape(input_shape), perp, enc_flat.reshape(_B, _K))


# speedup vs baseline: 1.7496x; 1.7496x over previous
import jax
import jax.numpy as jnp
from jax.experimental import pallas as pl
from jax.experimental.pallas import tpu as pltpu

_B = 16384
_K = 1024
_D = 256
_R = 2048
_COMMITMENT_COST = 0.25


def _vq_block_kernel(x_ref, sx_ref, se_ref, w_ref,
                     enc_ref, q_ref, cnt_ref, ds_ref):
    x = x_ref[...]
    w = w_ref[...]
    m = jax.lax.dot_general(x, w, (((1,), (1,)), ((), ())),
                            preferred_element_type=jnp.float32)
    d = (sx_ref[...] + se_ref[...]) - 2.0 * m
    dmin = jnp.min(d, axis=1, keepdims=True)
    cols = jax.lax.broadcasted_iota(jnp.int32, (d.shape[0], d.shape[1]), 1)
    idx = jnp.min(jnp.where(d == dmin, cols, _K), axis=1, keepdims=True)
    onehot = (cols == idx).astype(jnp.float32)
    enc_ref[...] = onehot
    q = jax.lax.dot_general(onehot, w, (((1,), (0,)), ((), ())),
                            preferred_element_type=jnp.float32)
    q_ref[...] = x + (q - x)
    ones = jnp.ones((1, x.shape[0]), jnp.float32)
    cnt = jax.lax.dot_general(ones, onehot, (((1,), (0,)), ((), ())),
                              preferred_element_type=jnp.float32)
    cnt_ref[...] = cnt.reshape(1, 1, _K)
    ds_ref[...] = jnp.sum(dmin).reshape(1, 1, 1)


def kernel(inputs, classes, embeddings_weight):
    del classes
    input_shape = inputs.shape
    x = inputs.reshape(_B, _D)
    sx = jnp.sum(x ** 2, axis=1, keepdims=True)
    se = jnp.sum(embeddings_weight ** 2, axis=1)[None, :]
    grid = _B // _R
    enc, q, cnt, ds = pl.pallas_call(
        _vq_block_kernel,
        grid=(grid,),
        in_specs=[
            pl.BlockSpec((_R, _D), lambda i: (i, 0)),
            pl.BlockSpec((_R, 1), lambda i: (i, 0)),
            pl.BlockSpec((1, _K), lambda i: (0, 0)),
            pl.BlockSpec((_K, _D), lambda i: (0, 0)),
        ],
        out_specs=[
            pl.BlockSpec((_R, _K), lambda i: (i, 0)),
            pl.BlockSpec((_R, _D), lambda i: (i, 0)),
            pl.BlockSpec((1, 1, _K), lambda i: (i, 0, 0)),
            pl.BlockSpec((1, 1, 1), lambda i: (i, 0, 0)),
        ],
        out_shape=[
            jax.ShapeDtypeStruct((_B, _K), jnp.float32),
            jax.ShapeDtypeStruct((_B, _D), jnp.float32),
            jax.ShapeDtypeStruct((grid, 1, _K), jnp.float32),
            jax.ShapeDtypeStruct((grid, 1, 1), jnp.float32),
        ],
        compiler_params=pltpu.CompilerParams(
            dimension_semantics=("parallel",)),
    )(x, sx, se, embeddings_weight)
    loss = (1.0 + _COMMITMENT_COST) * jnp.sum(ds) / (_B * _D)
    p = jnp.sum(cnt.reshape(grid, _K), axis=0) / _B
    perp = jnp.exp(-jnp.sum(p * jnp.log(p + 1e-10)))
    return (loss, q.reshape(input_shape), perp, enc)
